# Initial kernel scaffold; baseline (speedup 1.0000x reference)
import jax
import jax.numpy as jnp
from jax.experimental import pallas as pl


def kernel(hidden_states, Wq, Wk, Wv, Wo, Wg, Ck, Cv):
    raise NotImplementedError("write your pallas kernel here")


# speedup vs baseline: 1.5357x; 1.5357x over previous
import math

import jax
import jax.numpy as jnp
from jax.experimental import pallas as pl

B = 1
S = 2048
HID = 768
H = 12
HK = 2
G = H // HK
D = 64
KS = 32
STR = 16
BS = 64
TOPK = 16
INIT_B = 1
LOC_B = 2
WIN = 512
THETA = 10000.0
T = (S - KS) // STR + 1
TP = S // STR
NB = S // BS
SCALE = 1.0 / math.sqrt(D)
HALF = D // 2

PB = 256
QB = 128

_F32 = jnp.float32


def _rope_flat(x, ncols, cos1, sin1, swap64, sgn):
    parts = []
    for h in range(ncols // D):
        xh = x[:, h * D:(h + 1) * D]
        part = jnp.dot(xh, swap64, preferred_element_type=_F32)
        parts.append(xh * cos1 + part * (sgn * sin1))
    return jnp.concatenate(parts, axis=1)


def _proj_kernel(x_ref, wq_ref, wk_ref, wv_ref, wg_ref,
                 q_ref, k_ref, v_ref, g_ref):
    x = x_ref[...]
    q = jnp.dot(x, wq_ref[...], preferred_element_type=_F32)
    k = jnp.dot(x, wk_ref[...], preferred_element_type=_F32)
    v = jnp.dot(x, wv_ref[...], preferred_element_type=_F32)
    g = jax.nn.sigmoid(jnp.dot(x, wg_ref[...], preferred_element_type=_F32))
    base = pl.program_id(0) * PB
    pos = (base + jax.lax.broadcasted_iota(jnp.int32, (PB, 1), 0)).astype(_F32)
    r = jax.lax.broadcasted_iota(jnp.int32, (1, D), 1)
    r32 = r % HALF
    inv = jnp.exp(r32.astype(_F32) / HALF * (-math.log(THETA)))
    ang = pos * inv
    cos1 = jnp.cos(ang)
    sin1 = jnp.sin(ang)
    sgn = jnp.where(r < HALF, -1.0, 1.0)
    ri = jax.lax.broadcasted_iota(jnp.int32, (D, 1), 0)
    cj = jax.lax.broadcasted_iota(jnp.int32, (1, D), 1)
    swap64 = ((ri + HALF) % D == cj).astype(_F32)
    q_ref[...] = _rope_flat(q, H * D, cos1, sin1, swap64, sgn)
    k_ref[...] = _rope_flat(k, HK * D, cos1, sin1, swap64, sgn)
    v_ref[...] = v
    g_ref[...] = g


def _compress_kernel(kw_ref, vw_ref, ck_ref, cv_ref, ock_ref, ocv_ref):
    for h in range(HK):
        ock_ref[h] = jnp.dot(kw_ref[h], ck_ref[h], preferred_element_type=_F32)
        ocv_ref[h] = jnp.dot(vw_ref[h], cv_ref[h], preferred_element_type=_F32)


def _masked_av(s, mask, vh):
    sm = jnp.where(mask, s, -1e30)
    mx = jnp.max(sm, axis=-1, keepdims=True)
    p = jnp.exp(sm - mx) * mask.astype(_F32)
    dn = jnp.sum(p, axis=-1, keepdims=True)
    p = p / jnp.where(dn > 0.0, dn, 1.0)
    return p, jnp.dot(p, vh, preferred_element_type=_F32)


def _attn_kernel(q_ref, k_ref, v_ref, ck_ref, cv_ref, g_ref, wo_ref, o_ref):
    base = pl.program_id(0) * QB
    qpos = base + jax.lax.broadcasted_iota(jnp.int32, (QB, 1), 0)
    gates = g_ref[...]

    tidx = jax.lax.broadcasted_iota(jnp.int32, (1, TP), 1)
    allowed_c = qpos >= (STR * tidx + KS - 1)
    kpos = jax.lax.broadcasted_iota(jnp.int32, (1, S), 1)
    causal = qpos >= kpos
    win_m = causal & ((qpos - kpos) <= WIN)
    qblk = qpos // BS
    nb = jax.lax.broadcasted_iota(jnp.int32, (1, NB), 1)
    forced = (nb < INIT_B) | ((nb <= qblk) & (nb >= qblk - (LOC_B - 1)))
    causal_b = nb <= qblk
    ti = jax.lax.broadcasted_iota(jnp.int32, (TP, 1), 0)
    m4 = ((ti // (BS // STR)) == nb).astype(_F32)
    nbi = jax.lax.broadcasted_iota(jnp.int32, (NB, 1), 0)
    expand = (nbi == (kpos // BS)).astype(_F32)
    jidx = jax.lax.broadcasted_iota(jnp.int32, (1, NB), 1)

    g0 = gates[:, 0:1]
    g1 = gates[:, 1:2]
    g2 = gates[:, 2:3]

    outs = []
    for hk in range(HK):
        ckh = ck_ref[hk]
        cvh = cv_ref[hk]
        kh = k_ref[...][:, hk * D:(hk + 1) * D]
        vh = v_ref[...][:, hk * D:(hk + 1) * D]
        psum = jnp.zeros((QB, TP), _F32)
        qhs = []
        cmps = []
        for g in range(G):
            hh = hk * G + g
            qh = q_ref[...][:, hh * D:(hh + 1) * D]
            qhs.append(qh)
            sc = jax.lax.dot_general(qh, ckh, (((1,), (1,)), ((), ())),
                                     preferred_element_type=_F32) * SCALE
            pc, cmp = _masked_av(sc, allowed_c, cvh)
            cmps.append(cmp)
            psum = psum + pc
        blk = jnp.dot(psum, m4, preferred_element_type=_F32)
        blkm = jnp.where(forced, 1e9, blk)
        blkm = jnp.where(causal_b, blkm, -1e30)
        cols = []
        for i in range(NB):
            vi = blkm[:, i:i + 1]
            beats = (blkm > vi) | ((blkm == vi) & (jidx < i))
            cols.append(jnp.sum(beats.astype(_F32), axis=1, keepdims=True))
        rank = jnp.concatenate(cols, axis=1)
        sel = (rank < TOPK) & causal_b
        sel_keys = jnp.dot(sel.astype(_F32), expand,
                           preferred_element_type=_F32) > 0.5
        spm = sel_keys & causal
        for g in range(G):
            s = jax.lax.dot_general(qhs[g], kh, (((1,), (1,)), ((), ())),
                                    preferred_element_type=_F32) * SCALE
            _, sp = _masked_av(s, spm, vh)
            _, sw = _masked_av(s, win_m, vh)
            outs.append(g0 * cmps[g] + g1 * sp + g2 * sw)

    comb = jnp.concatenate(outs, axis=1)
    o_ref[...] = jnp.dot(comb, wo_ref[...], preferred_element_type=_F32)


def kernel(hidden_states, Wq, Wk, Wv, Wo, Wg, Ck, Cv):
    x = hidden_states.reshape(S, HID)
    wq = Wq.T
    wk = Wk.T
    wv = Wv.T
    wo = Wo.T
    wg = jnp.concatenate([Wg.T, jnp.zeros((HID, 8 - Wg.shape[0]), _F32)], axis=1)

    q, k, v, g = pl.pallas_call(
        _proj_kernel,
        grid=(S // PB,),
        in_specs=[
            pl.BlockSpec((PB, HID), lambda i: (i, 0)),
            pl.BlockSpec((HID, H * D), lambda i: (0, 0)),
            pl.BlockSpec((HID, HK * D), lambda i: (0, 0)),
            pl.BlockSpec((HID, HK * D), lambda i: (0, 0)),
            pl.BlockSpec((HID, 8), lambda i: (0, 0)),
        ],
        out_specs=[
            pl.BlockSpec((PB, H * D), lambda i: (i, 0)),
            pl.BlockSpec((PB, HK * D), lambda i: (i, 0)),
            pl.BlockSpec((PB, HK * D), lambda i: (i, 0)),
            pl.BlockSpec((PB, 8), lambda i: (i, 0)),
        ],
        out_shape=[
            jax.ShapeDtypeStruct((S, H * D), _F32),
            jax.ShapeDtypeStruct((S, HK * D), _F32),
            jax.ShapeDtypeStruct((S, HK * D), _F32),
            jax.ShapeDtypeStruct((S, 8), _F32),
        ],
    )(x, wq, wk, wv, wg)

    def windows(a):
        rows = []
        for h in range(HK):
            ar = a[:, h * D:(h + 1) * D].reshape(S // STR, STR * D)
            w = jnp.concatenate([ar[:-1], ar[1:]], axis=1)
            rows.append(jnp.concatenate(
                [w, jnp.zeros((TP - T, KS * D), _F32)], axis=0))
        return jnp.stack(rows)

    ck, cv = pl.pallas_call(
        _compress_kernel,
        out_shape=[
            jax.ShapeDtypeStruct((HK, TP, D), _F32),
            jax.ShapeDtypeStruct((HK, TP, D), _F32),
        ],
    )(windows(k), windows(v), Ck, Cv)

    out = pl.pallas_call(
        _attn_kernel,
        grid=(S // QB,),
        in_specs=[
            pl.BlockSpec((QB, H * D), lambda i: (i, 0)),
            pl.BlockSpec((S, HK * D), lambda i: (0, 0)),
            pl.BlockSpec((S, HK * D), lambda i: (0, 0)),
            pl.BlockSpec((HK, TP, D), lambda i: (0, 0, 0)),
            pl.BlockSpec((HK, TP, D), lambda i: (0, 0, 0)),
            pl.BlockSpec((QB, 8), lambda i: (i, 0)),
            pl.BlockSpec((H * D, HID), lambda i: (0, 0)),
        ],
        out_specs=pl.BlockSpec((QB, HID), lambda i: (i, 0)),
        out_shape=jax.ShapeDtypeStruct((S, HID), _F32),
    )(q, k, v, ck, cv, g, wo)

    return out.reshape(B, S, HID)


# speedup vs baseline: 1.5633x; 1.0180x over previous
import math

import jax
import jax.numpy as jnp
from jax.experimental import pallas as pl
from jax.experimental.pallas import tpu as pltpu

B = 1
S = 2048
HID = 768
H = 12
HK = 2
G = H // HK
D = 64
KS = 32
STR = 16
BS = 64
TOPK = 16
INIT_B = 1
LOC_B = 2
WIN = 512
THETA = 10000.0
T = (S - KS) // STR + 1
TP = S // STR
NB = S // BS
SCALE = 1.0 / math.sqrt(D)
HALF = D // 2

PB = 256
QB = 128

_F32 = jnp.float32


def _rope_flat(x, ncols, cos1, sin1, swap64, sgn):
    parts = []
    for h in range(ncols // D):
        xh = x[:, h * D:(h + 1) * D]
        part = jnp.dot(xh, swap64, preferred_element_type=_F32)
        parts.append(xh * cos1 + part * (sgn * sin1))
    return jnp.concatenate(parts, axis=1)


def _proj_kernel(x_ref, wq_ref, wk_ref, wv_ref, wg_ref,
                 q_ref, k_ref, v_ref, g_ref):
    x = x_ref[...]
    q = jnp.dot(x, wq_ref[...], preferred_element_type=_F32)
    k = jnp.dot(x, wk_ref[...], preferred_element_type=_F32)
    v = jnp.dot(x, wv_ref[...], preferred_element_type=_F32)
    g = jax.nn.sigmoid(jnp.dot(x, wg_ref[...], preferred_element_type=_F32))
    base = pl.program_id(0) * PB
    pos = (base + jax.lax.broadcasted_iota(jnp.int32, (PB, 1), 0)).astype(_F32)
    r = jax.lax.broadcasted_iota(jnp.int32, (1, D), 1)
    r32 = r % HALF
    inv = jnp.exp(r32.astype(_F32) / HALF * (-math.log(THETA)))
    ang = pos * inv
    cos1 = jnp.cos(ang)
    sin1 = jnp.sin(ang)
    sgn = jnp.where(r < HALF, -1.0, 1.0)
    ri = jax.lax.broadcasted_iota(jnp.int32, (D, 1), 0)
    cj = jax.lax.broadcasted_iota(jnp.int32, (1, D), 1)
    swap64 = ((ri + HALF) % D == cj).astype(_F32)
    q_ref[...] = _rope_flat(q, H * D, cos1, sin1, swap64, sgn)
    k_ref[...] = _rope_flat(k, HK * D, cos1, sin1, swap64, sgn)
    v_ref[...] = v
    g_ref[...] = g


def _compress_kernel(kw_ref, vw_ref, ck_ref, cv_ref, ock_ref, ocv_ref):
    for h in range(HK):
        ock_ref[h] = jnp.dot(kw_ref[h], ck_ref[h], preferred_element_type=_F32)
        ocv_ref[h] = jnp.dot(vw_ref[h], cv_ref[h], preferred_element_type=_F32)


def _masked_av(s, mask, vh):
    sm = jnp.where(mask, s, -1e30)
    mx = jnp.max(sm, axis=-1, keepdims=True)
    p = jnp.exp(sm - mx) * mask.astype(_F32)
    dn = jnp.sum(p, axis=-1, keepdims=True)
    p = p / jnp.where(dn > 0.0, dn, 1.0)
    return p, jnp.dot(p, vh, preferred_element_type=_F32)


def _attn_kernel(q_ref, k_ref, v_ref, ck_ref, cv_ref, g_ref, wo_ref, o_ref):
    base = pl.program_id(0) * QB
    qpos = base + jax.lax.broadcasted_iota(jnp.int32, (QB, 1), 0)
    gates = g_ref[...]

    tidx = jax.lax.broadcasted_iota(jnp.int32, (1, TP), 1)
    allowed_c = qpos >= (STR * tidx + KS - 1)
    kpos = jax.lax.broadcasted_iota(jnp.int32, (1, S), 1)
    causal = qpos >= kpos
    win_m = causal & ((qpos - kpos) <= WIN)
    qblk = qpos // BS
    nb = jax.lax.broadcasted_iota(jnp.int32, (1, NB), 1)
    forced = (nb < INIT_B) | ((nb <= qblk) & (nb >= qblk - (LOC_B - 1)))
    causal_b = nb <= qblk
    ti = jax.lax.broadcasted_iota(jnp.int32, (TP, 1), 0)
    m4 = ((ti // (BS // STR)) == nb).astype(_F32)
    nbi = jax.lax.broadcasted_iota(jnp.int32, (NB, 1), 0)
    expand = (nbi == (kpos // BS)).astype(_F32)
    jidx = jax.lax.broadcasted_iota(jnp.int32, (1, NB), 1)

    g0 = gates[:, 0:1]
    g1 = gates[:, 1:2]
    g2 = gates[:, 2:3]

    outs = []
    for hk in range(HK):
        ckh = ck_ref[hk]
        cvh = cv_ref[hk]
        kh = k_ref[...][:, hk * D:(hk + 1) * D]
        vh = v_ref[...][:, hk * D:(hk + 1) * D]
        psum = jnp.zeros((QB, TP), _F32)
        qhs = []
        cmps = []
        for g in range(G):
            hh = hk * G + g
            qh = q_ref[...][:, hh * D:(hh + 1) * D]
            qhs.append(qh)
            sc = jax.lax.dot_general(qh, ckh, (((1,), (1,)), ((), ())),
                                     preferred_element_type=_F32) * SCALE
            pc, cmp = _masked_av(sc, allowed_c, cvh)
            cmps.append(cmp)
            psum = psum + pc
        blk = jnp.dot(psum, m4, preferred_element_type=_F32)
        blkm = jnp.where(forced, 1e9, blk)
        blkm = jnp.where(causal_b, blkm, -1e30)
        cols = []
        for i in range(NB):
            vi = blkm[:, i:i + 1]
            beats = (blkm > vi) | ((blkm == vi) & (jidx < i))
            cols.append(jnp.sum(beats.astype(_F32), axis=1, keepdims=True))
        rank = jnp.concatenate(cols, axis=1)
        sel = (rank < TOPK) & causal_b
        sel_keys = jnp.dot(sel.astype(_F32), expand,
                           preferred_element_type=_F32) > 0.5
        spm = sel_keys & causal
        for g in range(G):
            s = jax.lax.dot_general(qhs[g], kh, (((1,), (1,)), ((), ())),
                                    preferred_element_type=_F32) * SCALE
            _, sp = _masked_av(s, spm, vh)
            _, sw = _masked_av(s, win_m, vh)
            outs.append(g0 * cmps[g] + g1 * sp + g2 * sw)

    comb = jnp.concatenate(outs, axis=1)
    o_ref[...] = jnp.dot(comb, wo_ref[...], preferred_element_type=_F32)


def kernel(hidden_states, Wq, Wk, Wv, Wo, Wg, Ck, Cv):
    x = hidden_states.reshape(S, HID)
    wq = Wq.T
    wk = Wk.T
    wv = Wv.T
    wo = Wo.T
    wg = jnp.concatenate([Wg.T, jnp.zeros((HID, 8 - Wg.shape[0]), _F32)], axis=1)

    q, k, v, g = pl.pallas_call(
        _proj_kernel,
        grid=(S // PB,),
        in_specs=[
            pl.BlockSpec((PB, HID), lambda i: (i, 0)),
            pl.BlockSpec((HID, H * D), lambda i: (0, 0)),
            pl.BlockSpec((HID, HK * D), lambda i: (0, 0)),
            pl.BlockSpec((HID, HK * D), lambda i: (0, 0)),
            pl.BlockSpec((HID, 8), lambda i: (0, 0)),
        ],
        out_specs=[
            pl.BlockSpec((PB, H * D), lambda i: (i, 0)),
            pl.BlockSpec((PB, HK * D), lambda i: (i, 0)),
            pl.BlockSpec((PB, HK * D), lambda i: (i, 0)),
            pl.BlockSpec((PB, 8), lambda i: (i, 0)),
        ],
        out_shape=[
            jax.ShapeDtypeStruct((S, H * D), _F32),
            jax.ShapeDtypeStruct((S, HK * D), _F32),
            jax.ShapeDtypeStruct((S, HK * D), _F32),
            jax.ShapeDtypeStruct((S, 8), _F32),
        ],
        compiler_params=pltpu.CompilerParams(
            dimension_semantics=("parallel",)),
    )(x, wq, wk, wv, wg)

    def windows(a):
        rows = []
        for h in range(HK):
            ar = a[:, h * D:(h + 1) * D].reshape(S // STR, STR * D)
            w = jnp.concatenate([ar[:-1], ar[1:]], axis=1)
            rows.append(jnp.concatenate(
                [w, jnp.zeros((TP - T, KS * D), _F32)], axis=0))
        return jnp.stack(rows)

    ck, cv = pl.pallas_call(
        _compress_kernel,
        out_shape=[
            jax.ShapeDtypeStruct((HK, TP, D), _F32),
            jax.ShapeDtypeStruct((HK, TP, D), _F32),
        ],
    )(windows(k), windows(v), Ck, Cv)

    out = pl.pallas_call(
        _attn_kernel,
        grid=(S // QB,),
        in_specs=[
            pl.BlockSpec((QB, H * D), lambda i: (i, 0)),
            pl.BlockSpec((S, HK * D), lambda i: (0, 0)),
            pl.BlockSpec((S, HK * D), lambda i: (0, 0)),
            pl.BlockSpec((HK, TP, D), lambda i: (0, 0, 0)),
            pl.BlockSpec((HK, TP, D), lambda i: (0, 0, 0)),
            pl.BlockSpec((QB, 8), lambda i: (i, 0)),
            pl.BlockSpec((H * D, HID), lambda i: (0, 0)),
        ],
        out_specs=pl.BlockSpec((QB, HID), lambda i: (i, 0)),
        out_shape=jax.ShapeDtypeStruct((S, HID), _F32),
        compiler_params=pltpu.CompilerParams(
            dimension_semantics=("parallel",)),
    )(q, k, v, ck, cv, g, wo)

    return out.reshape(B, S, HID)


# speedup vs baseline: 1.6405x; 1.0494x over previous
import math

import jax
import jax.numpy as jnp
from jax.experimental import pallas as pl
from jax.experimental.pallas import tpu as pltpu

B = 1
S = 2048
HID = 768
H = 12
HK = 2
G = H // HK
D = 64
KS = 32
STR = 16
BS = 64
TOPK = 16
INIT_B = 1
LOC_B = 2
WIN = 512
THETA = 10000.0
T = (S - KS) // STR + 1
TP = S // STR
NB = S // BS
SCALE = 1.0 / math.sqrt(D)
HALF = D // 2

PB = 256
QB = 128

_F32 = jnp.float32


def _rope_flat(x, ncols, cos1, sin1, swap64, sgn):
    parts = []
    for h in range(ncols // D):
        xh = x[:, h * D:(h + 1) * D]
        part = jnp.dot(xh, swap64, preferred_element_type=_F32)
        parts.append(xh * cos1 + part * (sgn * sin1))
    return jnp.concatenate(parts, axis=1)


def _proj_kernel(x_ref, wq_ref, wk_ref, wv_ref, wg_ref,
                 q_ref, k_ref, v_ref, g_ref):
    x = x_ref[...]
    q = jnp.dot(x, wq_ref[...], preferred_element_type=_F32)
    k = jnp.dot(x, wk_ref[...], preferred_element_type=_F32)
    v = jnp.dot(x, wv_ref[...], preferred_element_type=_F32)
    g = jax.nn.sigmoid(jnp.dot(x, wg_ref[...], preferred_element_type=_F32))
    base = pl.program_id(0) * PB
    pos = (base + jax.lax.broadcasted_iota(jnp.int32, (PB, 1), 0)).astype(_F32)
    r = jax.lax.broadcasted_iota(jnp.int32, (1, D), 1)
    r32 = r % HALF
    inv = jnp.exp(r32.astype(_F32) / HALF * (-math.log(THETA)))
    ang = pos * inv
    cos1 = jnp.cos(ang)
    sin1 = jnp.sin(ang)
    sgn = jnp.where(r < HALF, -1.0, 1.0)
    ri = jax.lax.broadcasted_iota(jnp.int32, (D, 1), 0)
    cj = jax.lax.broadcasted_iota(jnp.int32, (1, D), 1)
    swap64 = ((ri + HALF) % D == cj).astype(_F32)
    q_ref[...] = _rope_flat(q, H * D, cos1, sin1, swap64, sgn)
    k_ref[...] = _rope_flat(k, HK * D, cos1, sin1, swap64, sgn)
    v_ref[...] = v
    g_ref[...] = g


def _compress_kernel(kw_ref, vw_ref, ck_ref, cv_ref, ock_ref, ocv_ref):
    for h in range(HK):
        ock_ref[h] = jnp.dot(kw_ref[h], ck_ref[h], preferred_element_type=_F32)
        ocv_ref[h] = jnp.dot(vw_ref[h], cv_ref[h], preferred_element_type=_F32)


def _masked_av(s, mask, vh):
    sm = jnp.where(mask, s, -1e30)
    mx = jnp.max(sm, axis=-1, keepdims=True)
    p = jnp.exp(sm - mx) * mask.astype(_F32)
    dn = jnp.sum(p, axis=-1, keepdims=True)
    p = p / jnp.where(dn > 0.0, dn, 1.0)
    return p, jnp.dot(p, vh, preferred_element_type=_F32)


CK = 256
WINW = WIN + QB


def _attn_kernel(q_ref, k_ref, v_ref, ck_ref, cv_ref, g_ref, wo_ref, o_ref):
    pid = pl.program_id(0)
    base = pid * QB
    qpos = base + jax.lax.broadcasted_iota(jnp.int32, (QB, 1), 0)
    gates = g_ref[...]

    tidx = jax.lax.broadcasted_iota(jnp.int32, (1, TP), 1)
    allowed_c = qpos >= (STR * tidx + KS - 1)
    qblk = qpos // BS
    nb = jax.lax.broadcasted_iota(jnp.int32, (1, NB), 1)
    forced = (nb < INIT_B) | ((nb <= qblk) & (nb >= qblk - (LOC_B - 1)))
    causal_b = nb <= qblk
    ti = jax.lax.broadcasted_iota(jnp.int32, (TP, 1), 0)
    m4 = ((ti // (BS // STR)) == nb).astype(_F32)
    nbi = jax.lax.broadcasted_iota(jnp.int32, (NB, 1), 0)
    jidx = jax.lax.broadcasted_iota(jnp.int32, (1, NB), 1)

    g0 = gates[:, 0:1]
    g1 = gates[:, 1:2]
    g2 = gates[:, 2:3]

    outs = []
    for hk in range(HK):
        ckh = ck_ref[hk]
        cvh = cv_ref[hk]
        psum = jnp.zeros((QB, TP), _F32)
        qhs = []
        cmps = []
        for g in range(G):
            hh = hk * G + g
            qh = q_ref[...][:, hh * D:(hh + 1) * D]
            qhs.append(qh)
            sc = jax.lax.dot_general(qh, ckh, (((1,), (1,)), ((), ())),
                                     preferred_element_type=_F32) * SCALE
            pc, cmp = _masked_av(sc, allowed_c, cvh)
            cmps.append(cmp)
            psum = psum + pc
        blk = jnp.dot(psum, m4, preferred_element_type=_F32)
        blkm = jnp.where(forced, 1e9, blk)
        blkm = jnp.where(causal_b, blkm, -1e30)
        cols = []
        for i in range(NB):
            vi = blkm[:, i:i + 1]
            beats = (blkm > vi) | ((blkm == vi) & (jidx < i))
            cols.append(jnp.sum(beats.astype(_F32), axis=1, keepdims=True))
        rank = jnp.concatenate(cols, axis=1)
        sel = (rank < TOPK) & causal_b
        sel_f = sel.astype(_F32)
        nchunks = pid // (CK // QB) + 1

        def chunk_body(j, carry):
            off = j * CK
            kc = k_ref[pl.ds(off, CK), hk * D:(hk + 1) * D]
            vc = v_ref[pl.ds(off, CK), hk * D:(hk + 1) * D]
            kpos_c = off + jax.lax.broadcasted_iota(jnp.int32, (1, CK), 1)
            exp_c = (nbi == (kpos_c // BS)).astype(_F32)
            selk = jnp.dot(sel_f, exp_c,
                           preferred_element_type=_F32) > 0.5
            mask = selk & (qpos >= kpos_c)
            maskf = mask.astype(_F32)
            new = []
            for g in range(G):
                m, l, acc = carry[g]
                s = jax.lax.dot_general(qhs[g], kc, (((1,), (1,)), ((), ())),
                                        preferred_element_type=_F32) * SCALE
                sm = jnp.where(mask, s, -1e30)
                m2 = jnp.maximum(m, jnp.max(sm, axis=-1, keepdims=True))
                p = jnp.exp(sm - m2) * maskf
                alpha = jnp.exp(m - m2)
                l2 = l * alpha + jnp.sum(p, axis=-1, keepdims=True)
                acc2 = acc * alpha + jnp.dot(p, vc, preferred_element_type=_F32)
                new.append((m2, l2, acc2))
            return tuple(new)

        init = tuple((jnp.full((QB, 1), -1e30, _F32),
                      jnp.zeros((QB, 1), _F32),
                      jnp.zeros((QB, D), _F32)) for _ in range(G))
        carr = jax.lax.fori_loop(0, nchunks, chunk_body, init)
        wstart = jnp.maximum(base + QB - WINW, 0)
        kwin = k_ref[pl.ds(wstart, WINW), hk * D:(hk + 1) * D]
        vwin = v_ref[pl.ds(wstart, WINW), hk * D:(hk + 1) * D]
        wpos = wstart + jax.lax.broadcasted_iota(jnp.int32, (1, WINW), 1)
        wmask = (qpos >= wpos) & ((qpos - wpos) <= WIN)
        for g in range(G):
            s = jax.lax.dot_general(qhs[g], kwin, (((1,), (1,)), ((), ())),
                                    preferred_element_type=_F32) * SCALE
            _, sw = _masked_av(s, wmask, vwin)
            m, l, acc = carr[g]
            sp = acc / jnp.where(l > 0.0, l, 1.0)
            outs.append(g0 * cmps[g] + g1 * sp + g2 * sw)

    comb = jnp.concatenate(outs, axis=1)
    o_ref[...] = jnp.dot(comb, wo_ref[...], preferred_element_type=_F32)


def kernel(hidden_states, Wq, Wk, Wv, Wo, Wg, Ck, Cv):
    x = hidden_states.reshape(S, HID)
    wq = Wq.T
    wk = Wk.T
    wv = Wv.T
    wo = Wo.T
    wg = jnp.concatenate([Wg.T, jnp.zeros((HID, 8 - Wg.shape[0]), _F32)], axis=1)

    q, k, v, g = pl.pallas_call(
        _proj_kernel,
        grid=(S // PB,),
        in_specs=[
            pl.BlockSpec((PB, HID), lambda i: (i, 0)),
            pl.BlockSpec((HID, H * D), lambda i: (0, 0)),
            pl.BlockSpec((HID, HK * D), lambda i: (0, 0)),
            pl.BlockSpec((HID, HK * D), lambda i: (0, 0)),
            pl.BlockSpec((HID, 8), lambda i: (0, 0)),
        ],
        out_specs=[
            pl.BlockSpec((PB, H * D), lambda i: (i, 0)),
            pl.BlockSpec((PB, HK * D), lambda i: (i, 0)),
            pl.BlockSpec((PB, HK * D), lambda i: (i, 0)),
            pl.BlockSpec((PB, 8), lambda i: (i, 0)),
        ],
        out_shape=[
            jax.ShapeDtypeStruct((S, H * D), _F32),
            jax.ShapeDtypeStruct((S, HK * D), _F32),
            jax.ShapeDtypeStruct((S, HK * D), _F32),
            jax.ShapeDtypeStruct((S, 8), _F32),
        ],
        compiler_params=pltpu.CompilerParams(
            dimension_semantics=("parallel",)),
    )(x, wq, wk, wv, wg)

    def windows(a):
        rows = []
        for h in range(HK):
            ar = a[:, h * D:(h + 1) * D].reshape(S // STR, STR * D)
            w = jnp.concatenate([ar[:-1], ar[1:]], axis=1)
            rows.append(jnp.concatenate(
                [w, jnp.zeros((TP - T, KS * D), _F32)], axis=0))
        return jnp.stack(rows)

    ck, cv = pl.pallas_call(
        _compress_kernel,
        out_shape=[
            jax.ShapeDtypeStruct((HK, TP, D), _F32),
            jax.ShapeDtypeStruct((HK, TP, D), _F32),
        ],
    )(windows(k), windows(v), Ck, Cv)

    out = pl.pallas_call(
        _attn_kernel,
        grid=(S // QB,),
        in_specs=[
            pl.BlockSpec((QB, H * D), lambda i: (i, 0)),
            pl.BlockSpec((S, HK * D), lambda i: (0, 0)),
            pl.BlockSpec((S, HK * D), lambda i: (0, 0)),
            pl.BlockSpec((HK, TP, D), lambda i: (0, 0, 0)),
            pl.BlockSpec((HK, TP, D), lambda i: (0, 0, 0)),
            pl.BlockSpec((QB, 8), lambda i: (i, 0)),
            pl.BlockSpec((H * D, HID), lambda i: (0, 0)),
        ],
        out_specs=pl.BlockSpec((QB, HID), lambda i: (i, 0)),
        out_shape=jax.ShapeDtypeStruct((S, HID), _F32),
        compiler_params=pltpu.CompilerParams(
            dimension_semantics=("parallel",)),
    )(q, k, v, ck, cv, g, wo)

    return out.reshape(B, S, HID)
